# baseline (speedup 1.0000x reference)
import jax
import jax.numpy as jnp
from jax import lax
from jax.experimental import pallas as pl
from jax.experimental.pallas import tpu as pltpu
from jax.experimental.pallas import tpu_sc as plsc

VOCAB = 1000000
EMBED_DIM = 64
NC, NS = 2, 16

_mesh = plsc.VectorSubcoreMesh(core_axis_name="c", subcore_axis_name="s",
                               num_cores=NC, num_subcores=NS)


def _tiny_body(idx_hbm, out_hbm, idx_v, osem0):
    wid = lax.axis_index("s") * NC + lax.axis_index("c")
    pltpu.sync_copy(idx_hbm.at[0], idx_v)
    pltpu.async_copy(idx_v, out_hbm.at[wid], osem0).wait()


_tiny = pl.kernel(
    _tiny_body,
    out_type=jax.ShapeDtypeStruct((32, 128), jnp.int32),
    mesh=_mesh,
    scratch_types=[
        pltpu.VMEM((128,), jnp.int32),
        pltpu.SemaphoreType.DMA,
    ],
    compiler_params=pltpu.CompilerParams(use_tc_tiling_on_sc=False),
)

_RB = 8000


def _copy_body(a_ref, o_ref):
    o_ref[...] = a_ref[...]


_copy = pl.pallas_call(
    _copy_body,
    out_shape=jax.ShapeDtypeStruct((VOCAB, EMBED_DIM), jnp.float32),
    grid=(VOCAB // _RB,),
    in_specs=[pl.BlockSpec((_RB, EMBED_DIM), lambda i: (i, 0))],
    out_specs=pl.BlockSpec((_RB, EMBED_DIM), lambda i: (i, 0)),
)


def kernel(input, weight):
    token = _tiny(input.reshape(1600, 128).astype(jnp.int32))
    w2 = _copy(weight)
    return w2[0, 0] + jnp.float32(token[0, 0]), w2


# speedup vs baseline: 1.4550x; 1.4550x over previous
import jax
import jax.numpy as jnp
from jax import lax
from jax.experimental import pallas as pl
from jax.experimental.pallas import tpu as pltpu
from jax.experimental.pallas import tpu_sc as plsc

VOCAB = 1000000
EMBED_DIM = 64
BATCH = 4096
HIST = 50

NC, NS = 2, 16
NW = NC * NS
B_CH = 8
CHUNK = B_CH * HIST
N_IDX = BATCH * HIST
CPW = N_IDX // (NW * CHUNK)
BPW = BATCH // NW
HP = 56
DP = 128

_mesh = plsc.VectorSubcoreMesh(core_axis_name="c", subcore_axis_name="s",
                               num_cores=NC, num_subcores=NS)


def _body(idx_hbm, tbl_hbm, out_hbm, idx_v, rows0, rows1, gsem0, gsem1,
          osem0, osem1):
    wid = lax.axis_index("s") * NC + lax.axis_index("c")
    bbase = wid * BPW
    pltpu.sync_copy(idx_hbm.at[wid], idx_v)

    rows = (rows0, rows1)
    gsem = (gsem0, gsem1)
    osem = (osem0, osem1)

    def gather(j, b):
        return pltpu.async_copy(tbl_hbm.at[idx_v.at[j]], rows[b], gsem[b])

    def outcopy(j, b):
        descs = []
        for k in range(B_CH):
            descs.append(pltpu.async_copy(
                rows[b].at[pl.ds(k * HIST, HIST)],
                out_hbm.at[bbase + j * B_CH + k, pl.ds(0, HIST),
                           pl.ds(0, EMBED_DIM)],
                osem[b]))
        return descs

    def wait_all(descs):
        for d in descs:
            d.wait()

    g = [None, None]
    o = [None, None]
    g[0] = gather(0, 0)
    for j in range(CPW):
        b, nb = j % 2, (j + 1) % 2
        if j + 1 < CPW:
            if o[nb] is not None:
                wait_all(o[nb])
            g[nb] = gather(j + 1, nb)
        g[b].wait()
        o[b] = outcopy(j, b)
    wait_all(o[0])
    wait_all(o[1])


_gather = pl.kernel(
    _body,
    out_type=jax.ShapeDtypeStruct((BATCH, HP, DP), jnp.float32),
    mesh=_mesh,
    scratch_types=[
        pltpu.VMEM((CPW, CHUNK), jnp.int32),
        pltpu.VMEM((CHUNK, EMBED_DIM), jnp.float32),
        pltpu.VMEM((CHUNK, EMBED_DIM), jnp.float32),
        pltpu.SemaphoreType.DMA,
        pltpu.SemaphoreType.DMA,
        pltpu.SemaphoreType.DMA,
        pltpu.SemaphoreType.DMA,
    ],
    compiler_params=pltpu.CompilerParams(use_tc_tiling_on_sc=False),
)


def kernel(input, weight):
    idx = input.reshape(NW, CPW, CHUNK).astype(jnp.int32)
    padded = _gather(idx, weight)
    return padded[:, :HIST, :EMBED_DIM]
